# initial kernel scaffold (unmeasured)
import jax
import jax.numpy as jnp
from jax import lax
from jax.experimental import pallas as pl
from jax.experimental.pallas import tpu as pltpu


def kernel(
    x,
):
    def body(*refs):
        pass

    out_shape = jax.ShapeDtypeStruct(..., jnp.float32)
    return pl.pallas_call(body, out_shape=out_shape)(...)


# baseline (device time: 225431 ns/iter reference)
import jax
import jax.numpy as jnp
from jax import lax
from jax.experimental import pallas as pl
from jax.experimental.pallas import tpu as pltpu

M = 8192
N = 2048
NH = N // 2
K = 8
R = M // K


def kernel(x):
    x_bf = x.astype(jnp.bfloat16)

    def body(x_ref, out_ref, recv_hbm, xbuf, rbuf,
             send_sems, recv_sems, sem_x, sem_r):
        my_p = lax.axis_index("x")
        my_y = lax.axis_index("y")
        my_z = lax.axis_index("z")
        peer = 1 - my_p

        barrier_sem = pltpu.get_barrier_semaphore()
        pl.semaphore_signal(
            barrier_sem,
            inc=1,
            device_id=(peer, my_y, my_z),
            device_id_type=pl.DeviceIdType.MESH,
        )
        pl.semaphore_wait(barrier_sem, 1)

        rdmas = []
        for k in range(K):
            rdma = pltpu.make_async_remote_copy(
                src_ref=x_ref.at[0, pl.ds(k * R, R), pl.ds(peer * NH, NH)],
                dst_ref=recv_hbm.at[pl.ds(k * R, R), :],
                send_sem=send_sems.at[k],
                recv_sem=recv_sems.at[k],
                device_id=(peer, my_y, my_z),
                device_id_type=pl.DeviceIdType.MESH,
            )
            rdma.start()
            rdmas.append(rdma)

        for k in range(K):
            slot = k % 2
            cx = pltpu.make_async_copy(
                x_ref.at[0, pl.ds(k * R, R), pl.ds(my_p * NH, NH)],
                xbuf.at[slot],
                sem_x.at[slot],
            )
            cx.start()
            rdmas[k].wait_recv()
            cr = pltpu.make_async_copy(
                recv_hbm.at[pl.ds(k * R, R), :],
                rbuf.at[slot],
                sem_r.at[slot],
            )
            cr.start()
            cx.wait()
            cr.wait()
            out_ref[pl.ds(k * R, R), :] = xbuf[slot] + rbuf[slot]

        for k in range(K):
            rdmas[k].wait_send()

    out, _ = pl.pallas_call(
        body,
        out_shape=[
            jax.ShapeDtypeStruct((M, NH), jnp.bfloat16),
            jax.ShapeDtypeStruct((M, NH), jnp.bfloat16),
        ],
        in_specs=[pl.BlockSpec(memory_space=pl.ANY)],
        out_specs=[
            pl.BlockSpec(memory_space=pltpu.VMEM),
            pl.BlockSpec(memory_space=pl.ANY),
        ],
        scratch_shapes=[
            pltpu.VMEM((2, R, NH), jnp.bfloat16),
            pltpu.VMEM((2, R, NH), jnp.bfloat16),
            pltpu.SemaphoreType.DMA((K,)),
            pltpu.SemaphoreType.DMA((K,)),
            pltpu.SemaphoreType.DMA((2,)),
            pltpu.SemaphoreType.DMA((2,)),
        ],
        compiler_params=pltpu.CompilerParams(collective_id=0),
    )(x_bf)
    return out


# device time: 192766 ns/iter; 1.1695x vs baseline; 1.1695x over previous
import jax
import jax.numpy as jnp
from jax import lax
from jax.experimental import pallas as pl
from jax.experimental.pallas import tpu as pltpu

M = 8192
N = 2048
NH = N // 2
K = 16
R = M // K
S = 4


def kernel(x):
    def body(x_ref, out_ref, recv_hbm, cvt_f32, send_buf, my_f32, rbuf,
             send_sems, recv_sems, peer_dma_sems, my_dma_sems, rbuf_sems):
        my_p = lax.axis_index("x")
        my_y = lax.axis_index("y")
        my_z = lax.axis_index("z")
        peer = 1 - my_p

        barrier_sem = pltpu.get_barrier_semaphore()
        pl.semaphore_signal(
            barrier_sem,
            inc=1,
            device_id=(peer, my_y, my_z),
            device_id_type=pl.DeviceIdType.MESH,
        )
        pl.semaphore_wait(barrier_sem, 1)

        def peer_chunk_dma(k):
            return pltpu.make_async_copy(
                x_ref.at[0, pl.ds(k * R, R), pl.ds(peer * NH, NH)],
                cvt_f32.at[k % 2],
                peer_dma_sems.at[k % 2],
            )

        def my_chunk_dma(k):
            return pltpu.make_async_copy(
                x_ref.at[0, pl.ds(k * R, R), pl.ds(my_p * NH, NH)],
                my_f32.at[k % 2],
                my_dma_sems.at[k % 2],
            )

        rdmas = []
        peer_chunk_dma(0).start()
        for k in range(K):
            if k + 1 < K:
                peer_chunk_dma(k + 1).start()
            peer_chunk_dma(k).wait()
            ss = k % S
            if k >= S:
                rdmas[k - S].wait_send()
            send_buf[ss] = cvt_f32[k % 2].astype(jnp.bfloat16)
            rdma = pltpu.make_async_remote_copy(
                src_ref=send_buf.at[ss],
                dst_ref=recv_hbm.at[pl.ds(k * R, R), :],
                send_sem=send_sems.at[k],
                recv_sem=recv_sems.at[k],
                device_id=(peer, my_y, my_z),
                device_id_type=pl.DeviceIdType.MESH,
            )
            rdma.start()
            rdmas.append(rdma)

        my_chunk_dma(0).start()
        for k in range(K):
            if k + 1 < K:
                my_chunk_dma(k + 1).start()
            rdmas[k].wait_recv()
            crb = pltpu.make_async_copy(
                recv_hbm.at[pl.ds(k * R, R), :],
                rbuf.at[k % 2],
                rbuf_sems.at[k % 2],
            )
            crb.start()
            my_chunk_dma(k).wait()
            crb.wait()
            out_ref[pl.ds(k * R, R), :] = (
                my_f32[k % 2] + rbuf[k % 2].astype(jnp.float32)
            ).astype(jnp.bfloat16)

        for k in range(K - S, K):
            rdmas[k].wait_send()

    out, _ = pl.pallas_call(
        body,
        out_shape=[
            jax.ShapeDtypeStruct((M, NH), jnp.bfloat16),
            jax.ShapeDtypeStruct((M, NH), jnp.bfloat16),
        ],
        in_specs=[pl.BlockSpec(memory_space=pl.ANY)],
        out_specs=[
            pl.BlockSpec(memory_space=pltpu.VMEM),
            pl.BlockSpec(memory_space=pl.ANY),
        ],
        scratch_shapes=[
            pltpu.VMEM((2, R, NH), jnp.float32),
            pltpu.VMEM((S, R, NH), jnp.bfloat16),
            pltpu.VMEM((2, R, NH), jnp.float32),
            pltpu.VMEM((2, R, NH), jnp.bfloat16),
            pltpu.SemaphoreType.DMA((K,)),
            pltpu.SemaphoreType.DMA((K,)),
            pltpu.SemaphoreType.DMA((2,)),
            pltpu.SemaphoreType.DMA((2,)),
            pltpu.SemaphoreType.DMA((2,)),
        ],
        compiler_params=pltpu.CompilerParams(collective_id=0),
    )(x)
    return out
